# baseline (speedup 1.0000x reference)
import jax
import jax.numpy as jnp
from jax.experimental import pallas as pl

_GRID = 8


def _copy_kernel(x_ref, o_ref):
    o_ref[...] = x_ref[...]


def kernel(x, queue):
    del queue
    rows = x.shape[0] // _GRID
    return pl.pallas_call(
        _copy_kernel,
        grid=(_GRID,),
        in_specs=[pl.BlockSpec((rows, x.shape[1]), lambda i: (i, 0))],
        out_specs=pl.BlockSpec((rows, x.shape[1]), lambda i: (i, 0)),
        out_shape=jax.ShapeDtypeStruct(x.shape, x.dtype),
    )(x)


# speedup vs baseline: 2.2080x; 2.2080x over previous
import jax
import jax.numpy as jnp
from jax.experimental import pallas as pl
from jax.experimental.pallas import tpu as pltpu

_N_CHUNKS = 4
_ROWS = 4096 // _N_CHUNKS


def _copy_kernel(x_ref, o_ref, scratch, in_sems, out_sems):
    for i in range(_N_CHUNKS):
        pltpu.make_async_copy(
            x_ref.at[pl.ds(i * _ROWS, _ROWS)], scratch.at[i], in_sems.at[i]
        ).start()
    for i in range(_N_CHUNKS):
        pltpu.make_async_copy(
            x_ref.at[pl.ds(i * _ROWS, _ROWS)], scratch.at[i], in_sems.at[i]
        ).wait()
        pltpu.make_async_copy(
            scratch.at[i], o_ref.at[pl.ds(i * _ROWS, _ROWS)], out_sems.at[i]
        ).start()
    for i in range(_N_CHUNKS):
        pltpu.make_async_copy(
            scratch.at[i], o_ref.at[pl.ds(i * _ROWS, _ROWS)], out_sems.at[i]
        ).wait()


def kernel(x, queue):
    del queue
    return pl.pallas_call(
        _copy_kernel,
        in_specs=[pl.BlockSpec(memory_space=pl.ANY)],
        out_specs=pl.BlockSpec(memory_space=pl.ANY),
        out_shape=jax.ShapeDtypeStruct(x.shape, x.dtype),
        scratch_shapes=[
            pltpu.VMEM((_N_CHUNKS, _ROWS, x.shape[1]), x.dtype),
            pltpu.SemaphoreType.DMA((_N_CHUNKS,)),
            pltpu.SemaphoreType.DMA((_N_CHUNKS,)),
        ],
    )(x)


# speedup vs baseline: 2.2375x; 1.0134x over previous
import jax
import jax.numpy as jnp
from jax.experimental import pallas as pl
from jax.experimental.pallas import tpu as pltpu

_N_CHUNKS = 8
_ROWS = 4096 // _N_CHUNKS


def _copy_kernel(x_ref, o_ref, scratch, in_sems, out_sems):
    for i in range(_N_CHUNKS):
        pltpu.make_async_copy(
            x_ref.at[pl.ds(i * _ROWS, _ROWS)], scratch.at[i], in_sems.at[i]
        ).start()
    for i in range(_N_CHUNKS):
        pltpu.make_async_copy(
            x_ref.at[pl.ds(i * _ROWS, _ROWS)], scratch.at[i], in_sems.at[i]
        ).wait()
        pltpu.make_async_copy(
            scratch.at[i], o_ref.at[pl.ds(i * _ROWS, _ROWS)], out_sems.at[i]
        ).start()
    for i in range(_N_CHUNKS):
        pltpu.make_async_copy(
            scratch.at[i], o_ref.at[pl.ds(i * _ROWS, _ROWS)], out_sems.at[i]
        ).wait()


def kernel(x, queue):
    del queue
    return pl.pallas_call(
        _copy_kernel,
        in_specs=[pl.BlockSpec(memory_space=pl.ANY)],
        out_specs=pl.BlockSpec(memory_space=pl.ANY),
        out_shape=jax.ShapeDtypeStruct(x.shape, x.dtype),
        scratch_shapes=[
            pltpu.VMEM((_N_CHUNKS, _ROWS, x.shape[1]), x.dtype),
            pltpu.SemaphoreType.DMA((_N_CHUNKS,)),
            pltpu.SemaphoreType.DMA((_N_CHUNKS,)),
        ],
    )(x)
